# baseline (speedup 1.0000x reference)
import jax
import jax.numpy as jnp
from jax import lax
from jax.experimental import pallas as pl
from jax.experimental.pallas import tpu as pltpu

_N = 10000
_D = 128
_BM = 400


def _prep_body(feat_ref, w_ref, b_ref, g_ref):
    f = feat_ref[...]
    b_ref[...] = lax.dot_general(
        f, w_ref[:, :_D],
        dimension_numbers=(((1,), (1,)), ((), ())),
        preferred_element_type=jnp.float32)
    g_ref[...] = lax.dot_general(
        f, w_ref[:, _D:],
        dimension_numbers=(((1,), (1,)), ((), ())),
        preferred_element_type=jnp.float32)


def _sage_body(adj_ref, g_ref, b_ref, out_ref):
    a = adj_ref[...]
    deg = jnp.sum(a, axis=1, keepdims=True)
    neigh = jnp.dot(a, g_ref[...], preferred_element_type=jnp.float32)
    out_ref[...] = b_ref[...] + neigh / (deg + 1.0)


def kernel(adj, features, W):
    b, g = pl.pallas_call(
        _prep_body,
        out_shape=(jax.ShapeDtypeStruct((_N, _D), jnp.float32),
                   jax.ShapeDtypeStruct((_N, _D), jnp.float32)),
    )(features, W)
    return pl.pallas_call(
        _sage_body,
        grid=(_N // _BM,),
        in_specs=[
            pl.BlockSpec((_BM, _N), lambda i: (i, 0)),
            pl.BlockSpec((_N, _D), lambda i: (0, 0)),
            pl.BlockSpec((_BM, _D), lambda i: (i, 0)),
        ],
        out_specs=pl.BlockSpec((_BM, _D), lambda i: (i, 0)),
        out_shape=jax.ShapeDtypeStruct((_N, _D), jnp.float32),
        compiler_params=pltpu.CompilerParams(
            dimension_semantics=("parallel",)),
    )(adj, g, b)
